# baseline (speedup 1.0000x reference)
import functools

import jax
import jax.numpy as jnp
from jax import lax
from jax.experimental import pallas as pl
from jax.experimental.pallas import tpu as pltpu
from jax.experimental.pallas import tpu_sc as plsc

B, N, F = 8, 512, 512
T = 7 * N
TBL_ROWS = B * 3 * N
NQ = 4
TQ = T // NQ
SLOTS = 12
CHUNK = 16
NCHUNK = TQ // CHUNK
VPB = N // 16
VPT = T // 16


def _body(tbl_hbm, dur_hbm, out_hbm, mask_hbm, dur_v, idx_full, mask_v, rows, zbuf, *sems):
    cid = lax.axis_index("c")
    sid = lax.axis_index("s")
    b = cid * 4 + sid // NQ
    q = sid % NQ

    pltpu.sync_copy(dur_hbm.at[b], dur_v)

    base_b = b * (3 * N)
    lane = lax.iota(jnp.int32, 16)

    def seg_body(i, carry):
        d = dur_v[pl.ds(i * 16, 16)]
        o = plsc.cumsum(d) - d + carry
        n = base_b + i * 16 + lane
        vmid = n + N
        for j in range(7):
            if j == 0:
                val = jnp.where(d >= 2, n, vmid)
            else:
                val = jnp.where(d == j + 1, n + 2 * N, vmid)
            plsc.store_scatter(idx_full, [o + j], val, mask=d > j)
        return carry + jnp.sum(d)
    total = lax.fori_loop(0, VPB, seg_body, jnp.int32(0))

    def patch_body(i, _):
        t16 = i * 16 + lane
        v = idx_full[pl.ds(i * 16, 16)]
        idx_full[pl.ds(i * 16, 16)] = jnp.where(t16 >= total, base_b, v)
        return _
    patch_hi = jnp.minimum(((total // CHUNK + 1) * CHUNK + 15) // 16, VPT)
    lax.fori_loop(total // 16, patch_hi, patch_body, 0)

    def toff(c):
        return (c * NQ + q) * CHUNK

    def live(c):
        return toff(c) < total

    def gather_start(c, s):
        @pl.when(live(c))
        def _():
            idxs = idx_full.at[pl.ds(toff(c), CHUNK)]
            pltpu.async_copy(tbl_hbm.at[idxs], rows.at[s], sems[s])

    def gather_wait(c, s):
        @pl.when(live(c))
        def _():
            idxs = idx_full.at[pl.ds(toff(c), CHUNK)]
            pltpu.make_async_copy(tbl_hbm.at[idxs], rows.at[s], sems[s]).wait()

    def store_start(c, s):
        dst = out_hbm.at[pl.ds(b * T + toff(c), CHUNK)]

        @pl.when(live(c))
        def _():
            pltpu.async_copy(rows.at[s], dst, sems[s])

        @pl.when(jnp.logical_not(live(c)))
        def _():
            pltpu.async_copy(zbuf, dst, sems[s])

    def store_wait(c, s):
        dst = out_hbm.at[pl.ds(b * T + toff(c), CHUNK)]
        pltpu.make_async_copy(zbuf, dst, sems[s]).wait()

    for s in range(SLOTS):
        gather_start(s, s)

    def zero_body(r, _):
        for k in range(F // 16):
            zbuf[r, pl.ds(k * 16, 16)] = jnp.zeros((16,), jnp.float32)
        return _
    lax.fori_loop(0, CHUNK, zero_body, 0)

    @pl.when(q == 0)
    def _():
        def mask_body(i, _):
            t16 = i * 16 + lane
            mask_v[pl.ds(i * 16, 16)] = jnp.where(t16 < total, 1, 0)
            return _
        lax.fori_loop(0, VPT, mask_body, 0)
        pltpu.sync_copy(mask_v, mask_hbm.at[b])

    def zero_tail(c, s):
        @pl.when(live(c))
        def _():
            lo = jnp.clip(total - toff(c), 0, CHUNK)

            def zrow(r, _):
                for k in range(F // 16):
                    rows[s, r, pl.ds(k * 16, 16)] = jnp.zeros((16,), jnp.float32)
                return _
            lax.fori_loop(lo, CHUNK, zrow, 0)

    for c in range(NCHUNK):
        s = c % SLOTS
        gather_wait(c, s)
        zero_tail(c, s)
        store_start(c, s)
        cr = c + 1
        if SLOTS <= cr < NCHUNK:
            sr = cr % SLOTS
            store_wait(cr - SLOTS, sr)
            gather_start(cr, sr)
    for c in range(NCHUNK - SLOTS, NCHUNK):
        store_wait(c, c % SLOTS)


@jax.jit
def _hmm_interp(table, durations):
    mesh = plsc.VectorSubcoreMesh(
        core_axis_name="c", subcore_axis_name="s", num_cores=2, num_subcores=16)
    run = pl.kernel(
        _body,
        out_type=(
            jax.ShapeDtypeStruct((B * T, F), jnp.float32),
            jax.ShapeDtypeStruct((B, T), jnp.int32),
        ),
        mesh=mesh,
        scratch_types=[
            pltpu.VMEM((N,), jnp.int32),
            pltpu.VMEM((T,), jnp.int32),
            pltpu.VMEM((T,), jnp.int32),
            pltpu.VMEM((SLOTS, CHUNK, F), jnp.float32),
            pltpu.VMEM((CHUNK, F), jnp.float32),
        ] + [pltpu.SemaphoreType.DMA] * SLOTS,
        compiler_params=pltpu.CompilerParams(needs_layout_passes=False),
    )
    return run(table, durations)


def kernel(start, mid, end, durations, max_frames):
    table = jnp.concatenate([start, mid, end], axis=1).reshape(B * 3 * N, F)
    out_flat, mask_i32 = _hmm_interp(table, durations)
    return out_flat.reshape(B, T, F), mask_i32.astype(jnp.bool_)


# speedup vs baseline: 1.1186x; 1.1186x over previous
import functools

import jax
import jax.numpy as jnp
from jax import lax
from jax.experimental import pallas as pl
from jax.experimental.pallas import tpu as pltpu
from jax.experimental.pallas import tpu_sc as plsc

B, N, F = 8, 512, 512
T = 7 * N
TBL_ROWS = B * 3 * N
NQ = 4
TQ = T // NQ
SLOTS = 6
CHUNK = 32
NCHUNK = TQ // CHUNK
VPB = N // 16
VPT = T // 16


def _body(tbl_hbm, dur_hbm, out_hbm, mask_hbm, dur_v, idx_full, mask_v, rows, zbuf, *sems):
    cid = lax.axis_index("c")
    sid = lax.axis_index("s")
    b = cid * 4 + sid // NQ
    q = sid % NQ

    pltpu.sync_copy(dur_hbm.at[b], dur_v)

    base_b = b * (3 * N)
    lane = lax.iota(jnp.int32, 16)

    def seg_body(i, carry):
        d = dur_v[pl.ds(i * 16, 16)]
        o = plsc.cumsum(d) - d + carry
        n = base_b + i * 16 + lane
        vmid = n + N
        for j in range(7):
            if j == 0:
                val = jnp.where(d >= 2, n, vmid)
            else:
                val = jnp.where(d == j + 1, n + 2 * N, vmid)
            plsc.store_scatter(idx_full, [o + j], val, mask=d > j)
        return carry + jnp.sum(d)
    total = lax.fori_loop(0, VPB, seg_body, jnp.int32(0))

    def patch_body(i, _):
        t16 = i * 16 + lane
        v = idx_full[pl.ds(i * 16, 16)]
        idx_full[pl.ds(i * 16, 16)] = jnp.where(t16 >= total, base_b, v)
        return _
    patch_hi = jnp.minimum(((total // CHUNK + 1) * CHUNK + 15) // 16, VPT)
    lax.fori_loop(total // 16, patch_hi, patch_body, 0)

    def toff(c):
        return (c * NQ + q) * CHUNK

    def live(c):
        return toff(c) < total

    def gather_start(c, s):
        @pl.when(live(c))
        def _():
            idxs = idx_full.at[pl.ds(toff(c), CHUNK)]
            pltpu.async_copy(tbl_hbm.at[idxs], rows.at[s], sems[s])

    def gather_wait(c, s):
        @pl.when(live(c))
        def _():
            idxs = idx_full.at[pl.ds(toff(c), CHUNK)]
            pltpu.make_async_copy(tbl_hbm.at[idxs], rows.at[s], sems[s]).wait()

    def store_start(c, s):
        dst = out_hbm.at[pl.ds(b * T + toff(c), CHUNK)]

        @pl.when(live(c))
        def _():
            pltpu.async_copy(rows.at[s], dst, sems[s])

        @pl.when(jnp.logical_not(live(c)))
        def _():
            pltpu.async_copy(zbuf, dst, sems[s])

    def store_wait(c, s):
        dst = out_hbm.at[pl.ds(b * T + toff(c), CHUNK)]
        pltpu.make_async_copy(zbuf, dst, sems[s]).wait()

    for s in range(SLOTS):
        gather_start(s, s)

    def zero_body(r, _):
        for k in range(F // 16):
            zbuf[r, pl.ds(k * 16, 16)] = jnp.zeros((16,), jnp.float32)
        return _
    lax.fori_loop(0, CHUNK, zero_body, 0)

    @pl.when(q == 0)
    def _():
        def mask_body(i, _):
            t16 = i * 16 + lane
            mask_v[pl.ds(i * 16, 16)] = jnp.where(t16 < total, 1, 0)
            return _
        lax.fori_loop(0, VPT, mask_body, 0)
        pltpu.sync_copy(mask_v, mask_hbm.at[b])

    def zero_tail(c, s):
        @pl.when(live(c))
        def _():
            lo = jnp.clip(total - toff(c), 0, CHUNK)

            def zrow(r, _):
                for k in range(F // 16):
                    rows[s, r, pl.ds(k * 16, 16)] = jnp.zeros((16,), jnp.float32)
                return _
            lax.fori_loop(lo, CHUNK, zrow, 0)

    for c in range(NCHUNK):
        s = c % SLOTS
        gather_wait(c, s)
        zero_tail(c, s)
        store_start(c, s)
        cr = c + 1
        if SLOTS <= cr < NCHUNK:
            sr = cr % SLOTS
            store_wait(cr - SLOTS, sr)
            gather_start(cr, sr)
    for c in range(NCHUNK - SLOTS, NCHUNK):
        store_wait(c, c % SLOTS)


@jax.jit
def _hmm_interp(table, durations):
    mesh = plsc.VectorSubcoreMesh(
        core_axis_name="c", subcore_axis_name="s", num_cores=2, num_subcores=16)
    run = pl.kernel(
        _body,
        out_type=(
            jax.ShapeDtypeStruct((B * T, F), jnp.float32),
            jax.ShapeDtypeStruct((B, T), jnp.int32),
        ),
        mesh=mesh,
        scratch_types=[
            pltpu.VMEM((N,), jnp.int32),
            pltpu.VMEM((T,), jnp.int32),
            pltpu.VMEM((T,), jnp.int32),
            pltpu.VMEM((SLOTS, CHUNK, F), jnp.float32),
            pltpu.VMEM((CHUNK, F), jnp.float32),
        ] + [pltpu.SemaphoreType.DMA] * SLOTS,
        compiler_params=pltpu.CompilerParams(needs_layout_passes=False),
    )
    return run(table, durations)


def kernel(start, mid, end, durations, max_frames):
    table = jnp.concatenate([start, mid, end], axis=1).reshape(B * 3 * N, F)
    out_flat, mask_i32 = _hmm_interp(table, durations)
    return out_flat.reshape(B, T, F), mask_i32.astype(jnp.bool_)
